# baseline (speedup 1.0000x reference)
import jax
import jax.numpy as jnp
from jax.experimental import pallas as pl
from jax.experimental.pallas import tpu as pltpu

_B = 8
_C0 = 32
_T0 = 256
_TP0 = 258
_N = _C0 * _TP0
_TILE_N = 1024
_T_OUT = 1024
_HALF = 512


def _linear_kernel(x_ref, w_ref, b_ref, o_ref):
    o_ref[...] = (
        jnp.dot(x_ref[...], w_ref[...], preferred_element_type=jnp.float32)
        + b_ref[...]
    )


def _decoder_kernel(hp_ref, wb0_ref, bb0_ref, U0_ref, wb1_ref, bb1_ref,
                    U1_ref, o_ref):
    hp = hp_ref[...]

    xcat = jnp.concatenate(
        [hp[:, 0:256], hp[:, 1:257], hp[:, 2:258]], axis=0)
    y = (jnp.dot(wb0_ref[...], xcat, preferred_element_type=jnp.float32)
         + bb0_ref[...])
    y = jax.nn.gelu(y, approximate=True)
    h = y + hp[:, 1:257]
    hp1 = jnp.dot(h, U0_ref[...], preferred_element_type=jnp.float32)

    xcat1 = jnp.concatenate(
        [hp1[:, 0:512], hp1[:, 2:514], hp1[:, 4:516]], axis=0)
    y1 = (jnp.dot(wb1_ref[...], xcat1, preferred_element_type=jnp.float32)
          + bb1_ref[...])
    y1 = jax.nn.gelu(y1, approximate=True)
    h1 = y1 + hp1[:, 2:514]

    o_ref[...] = jnp.dot(h1, U1_ref[...], preferred_element_type=jnp.float32)


def kernel(x, lin_wT, lin_b, wb_0, bb_0, U_0, wb_1, bb_1, U_1):
    L = x.shape[1]
    n_tiles = -(-_N // _TILE_N)

    h_full = pl.pallas_call(
        _linear_kernel,
        out_shape=jax.ShapeDtypeStruct((_B, _N), jnp.float32),
        grid=(n_tiles,),
        in_specs=[
            pl.BlockSpec((_B, L), lambda j: (0, 0)),
            pl.BlockSpec((L, _TILE_N), lambda j: (0, j)),
            pl.BlockSpec((1, _TILE_N), lambda j: (0, j)),
        ],
        out_specs=pl.BlockSpec((_B, _TILE_N), lambda j: (0, j)),
        compiler_params=pltpu.CompilerParams(
            dimension_semantics=("parallel",)),
    )(x, lin_wT, lin_b)

    hp = h_full.reshape(_B * _C0, _TP0)

    out2d = pl.pallas_call(
        _decoder_kernel,
        out_shape=jax.ShapeDtypeStruct((_B * _C0, _T_OUT), jnp.float32),
        grid=(2,),
        in_specs=[
            pl.BlockSpec((_B * _C0, _TP0), lambda j: (0, 0)),
            pl.BlockSpec(wb_0.shape, lambda j: (0, 0)),
            pl.BlockSpec(bb_0.shape, lambda j: (0, 0)),
            pl.BlockSpec(U_0.shape, lambda j: (0, 0)),
            pl.BlockSpec(wb_1.shape, lambda j: (0, 0)),
            pl.BlockSpec(bb_1.shape, lambda j: (0, 0)),
            pl.BlockSpec((U_1.shape[0], _HALF), lambda j: (0, j)),
        ],
        out_specs=pl.BlockSpec((_B * _C0, _HALF), lambda j: (0, j)),
        compiler_params=pltpu.CompilerParams(
            dimension_semantics=("parallel",)),
    )(hp, wb_0, bb_0, U_0, wb_1, bb_1, U_1)

    return out2d.reshape(_B, _C0, _T_OUT)


# speedup vs baseline: 1.0754x; 1.0754x over previous
import jax
import jax.numpy as jnp
from jax.experimental import pallas as pl
from jax.experimental.pallas import tpu as pltpu

_B = 8
_C0 = 32
_TP0 = 258
_N = _C0 * _TP0
_L = 1024
_KSTRIPS = 4
_KTILE = _L // _KSTRIPS
_T_OUT = 1024


def _fused_kernel(x_ref, w_ref, b_ref, wb0_ref, bb0_ref, U0_ref,
                  wb1_ref, bb1_ref, U1_ref, o_ref, acc_ref):
    j = pl.program_id(0)

    part = jnp.dot(x_ref[...], w_ref[...], preferred_element_type=jnp.float32)

    @pl.when(j == 0)
    def _init():
        acc_ref[...] = part + b_ref[...]

    @pl.when(j != 0)
    def _accum():
        acc_ref[...] += part

    @pl.when(j == _KSTRIPS - 1)
    def _decode():
        h_full = acc_ref[...]
        hp = jnp.concatenate(
            [h_full[b:b + 1, c * _TP0:(c + 1) * _TP0]
             for b in range(_B) for c in range(_C0)],
            axis=0,
        )

        xcat = jnp.concatenate(
            [hp[:, 0:256], hp[:, 1:257], hp[:, 2:258]], axis=0)
        y = (jnp.dot(wb0_ref[...], xcat, preferred_element_type=jnp.float32)
             + bb0_ref[...])
        y = jax.nn.gelu(y, approximate=True)
        h = y + hp[:, 1:257]
        hp1 = jnp.dot(h, U0_ref[...],
                      preferred_element_type=jnp.float32)

        xcat1 = jnp.concatenate(
            [hp1[:, 0:512], hp1[:, 2:514], hp1[:, 4:516]], axis=0)
        y1 = (jnp.dot(wb1_ref[...], xcat1, preferred_element_type=jnp.float32)
              + bb1_ref[...])
        y1 = jax.nn.gelu(y1, approximate=True)
        h1 = y1 + hp1[:, 2:514]

        o_ref[...] = jnp.dot(h1, U1_ref[...],
                             preferred_element_type=jnp.float32)


def kernel(x, lin_wT, lin_b, wb_0, bb_0, U_0, wb_1, bb_1, U_1):
    out2d = pl.pallas_call(
        _fused_kernel,
        out_shape=jax.ShapeDtypeStruct((_B * _C0, _T_OUT), jnp.float32),
        grid=(_KSTRIPS,),
        in_specs=[
            pl.BlockSpec((_B, _KTILE), lambda j: (0, j)),
            pl.BlockSpec((_KTILE, _N), lambda j: (j, 0)),
            pl.BlockSpec((1, _N), lambda j: (0, 0)),
            pl.BlockSpec(wb_0.shape, lambda j: (0, 0)),
            pl.BlockSpec(bb_0.shape, lambda j: (0, 0)),
            pl.BlockSpec(U_0.shape, lambda j: (0, 0)),
            pl.BlockSpec(wb_1.shape, lambda j: (0, 0)),
            pl.BlockSpec(bb_1.shape, lambda j: (0, 0)),
            pl.BlockSpec(U_1.shape, lambda j: (0, 0)),
        ],
        out_specs=pl.BlockSpec((_B * _C0, _T_OUT), lambda j: (0, 0)),
        scratch_shapes=[pltpu.VMEM((_B, _N), jnp.float32)],
        compiler_params=pltpu.CompilerParams(
            dimension_semantics=("arbitrary",)),
    )(x, lin_wT, lin_b, wb_0, bb_0, U_0, wb_1, bb_1, U_1)

    return out2d.reshape(_B, _C0, _T_OUT)
